# baseline (speedup 1.0000x reference)
import functools

import jax
import jax.numpy as jnp
from jax import lax
from jax.experimental import pallas as pl
from jax.experimental.pallas import tpu as pltpu
from jax.experimental.pallas import tpu_sc as plsc

EPS = 1e-5

NC, NS, L = 2, 16, 16
NW = NC * NS

B, S, F = 4, 8192, 768
R = B * S
ROWS_PER_W = R // NW
CHUNK = L
NCH = ROWS_PER_W // CHUNK
HALF = NCH // 2
NJ = F // L
UNROLL = 2


def _lane_sum(v):
    lanes = lax.iota(jnp.int32, L)
    for sh in (1, 2, 4, 8):
        v = v + v.at[lanes ^ sh].get(mode="promise_in_bounds")
    return v


def _rsqrt_newton(v):
    bits = lax.bitcast_convert_type(v, jnp.int32)
    y = lax.bitcast_convert_type(jnp.int32(0x5F3759DF) - (bits >> 1),
                                 jnp.float32)
    for _ in range(3):
        y = y * (1.5 - 0.5 * v * y * y)
    return y


def _sc_body(x_hbm, emb_hbm, gamma_hbm, beta_hbm, out_hbm,
             xc, ec, oc, gv, bv,
             sx0, sx1, se0, se1, so0, so1):
    wid = lax.axis_index("s") * NC + lax.axis_index("c")
    row0 = wid * ROWS_PER_W
    erow0 = lax.rem(row0, S)

    pltpu.sync_copy(gamma_hbm, gv)
    pltpu.sync_copy(beta_hbm, bv)

    sx = (sx0, sx1)
    se = (se0, se1)
    so = (so0, so1)

    def xin(c, b):
        return pltpu.make_async_copy(
            x_hbm.at[pl.ds(row0 + c * CHUNK, CHUNK)], xc.at[b], sx[b])

    def ein(c, b):
        return pltpu.make_async_copy(
            emb_hbm.at[pl.ds(erow0 + c * CHUNK, CHUNK)], ec.at[b], se[b])

    def oout(c, b):
        return pltpu.make_async_copy(
            oc.at[b], out_hbm.at[pl.ds(row0 + c * CHUNK, CHUNK)], so[b])

    for b in (0, 1):
        xin(b, b).start()
        ein(b, b).start()

    zeros = tuple(jnp.zeros((L,), jnp.float32) for _ in range(CHUNK))

    def loop_body(i, _):
        for b in (0, 1):
            c = 2 * i + b
            xcb, ecb, ocb = xc.at[b], ec.at[b], oc.at[b]

            @pl.when(i > 0)
            def _():
                oout(c - 2, b).wait()

            xin(c, b).wait()
            ein(c, b).wait()

            def p1(j, carry):
                ss, qq = carry
                for u in range(UNROLL):
                    sl = pl.ds((UNROLL * j + u) * L, L)
                    nss, nqq = [], []
                    for r in range(CHUNK):
                        h = xcb[r, sl] + ecb[r, sl]
                        ocb[r, sl] = h
                        nss.append(ss[r] + h)
                        nqq.append(qq[r] + h * h)
                    ss, qq = tuple(nss), tuple(nqq)
                return ss, qq

            ss, qq = lax.fori_loop(0, NJ // UNROLL, p1, (zeros, zeros))

            @pl.when(i < HALF - 1)
            def _():
                xin(c + 2, b).start()
                ein(c + 2, b).start()

            A, D = [], []
            for r in range(CHUNK):
                mv = _lane_sum(ss[r]) * (1.0 / F)
                var = _lane_sum(qq[r]) * (1.0 / F) - mv * mv + EPS
                rs = _rsqrt_newton(var)
                A.append(rs)
                D.append(mv * rs)

            def p2(j, _):
                for u in range(UNROLL):
                    sl = pl.ds((UNROLL * j + u) * L, L)
                    g = gv[sl]
                    bt = bv[sl]
                    for r in range(CHUNK):
                        h = ocb[r, sl]
                        ocb[r, sl] = (h * A[r] - D[r]) * g + bt
                return 0

            lax.fori_loop(0, NJ // UNROLL, p2, 0)
            oout(c, b).start()
        return 0

    lax.fori_loop(0, HALF, loop_body, 0)
    oout(NCH - 2, 0).wait()
    oout(NCH - 1, 1).wait()


_sc_kernel = functools.partial(
    pl.kernel,
    out_type=jax.ShapeDtypeStruct((R, F), jnp.float32),
    mesh=plsc.VectorSubcoreMesh(core_axis_name="c", subcore_axis_name="s"),
    scratch_types=[
        pltpu.VMEM((2, CHUNK, F), jnp.float32),
        pltpu.VMEM((2, CHUNK, F), jnp.float32),
        pltpu.VMEM((2, CHUNK, F), jnp.float32),
        pltpu.VMEM((F,), jnp.float32),
        pltpu.VMEM((F,), jnp.float32),
        pltpu.SemaphoreType.DMA,
        pltpu.SemaphoreType.DMA,
        pltpu.SemaphoreType.DMA,
        pltpu.SemaphoreType.DMA,
        pltpu.SemaphoreType.DMA,
        pltpu.SemaphoreType.DMA,
    ],
)(_sc_body)


def kernel(x, emb, gamma, beta):
    b, s, f = x.shape
    out = _sc_kernel(x.reshape(b * s, f), emb[:s], gamma, beta)
    return out.reshape(b, s, f)


# speedup vs baseline: 2.5211x; 2.5211x over previous
import functools

import jax
import jax.numpy as jnp
from jax import lax
from jax.experimental import pallas as pl
from jax.experimental.pallas import tpu as pltpu
from jax.experimental.pallas import tpu_sc as plsc

EPS = 1e-5

NC, NS, L = 2, 16, 16
NW = NC * NS

B, S, F = 4, 8192, 768
R = B * S
ROWS_PER_W = R // NW
CHUNK = L
NCH = ROWS_PER_W // CHUNK
HALF = NCH // 2
NJ = F // L
UNROLL = 1


def _lane_sum(v):
    lanes = lax.iota(jnp.int32, L)
    for sh in (1, 2, 4, 8):
        v = v + v.at[lanes ^ sh].get(mode="promise_in_bounds")
    return v


def _rsqrt_newton(v):
    bits = lax.bitcast_convert_type(v, jnp.int32)
    y = lax.bitcast_convert_type(jnp.int32(0x5F3759DF) - (bits >> 1),
                                 jnp.float32)
    for _ in range(3):
        y = y * (1.5 - 0.5 * v * y * y)
    return y


def _sc_body(x_hbm, emb_hbm, gamma_hbm, beta_hbm, out_hbm,
             xc, ec, oc, gv, bv,
             sx0, sx1, se0, se1, so0, so1):
    wid = lax.axis_index("s") * NC + lax.axis_index("c")
    row0 = wid * ROWS_PER_W
    erow0 = lax.rem(row0, S)

    pltpu.sync_copy(gamma_hbm, gv)
    pltpu.sync_copy(beta_hbm, bv)

    sx = (sx0, sx1)
    se = (se0, se1)
    so = (so0, so1)

    def xin(c, b):
        return pltpu.make_async_copy(
            x_hbm.at[pl.ds(row0 + c * CHUNK, CHUNK)], xc.at[b], sx[b])

    def ein(c, b):
        return pltpu.make_async_copy(
            emb_hbm.at[pl.ds(erow0 + c * CHUNK, CHUNK)], ec.at[b], se[b])

    def oout(c, b):
        return pltpu.make_async_copy(
            oc.at[b], out_hbm.at[pl.ds(row0 + c * CHUNK, CHUNK)], so[b])

    for b in (0, 1):
        xin(b, b).start()
        ein(b, b).start()

    zeros = tuple(jnp.zeros((L,), jnp.float32) for _ in range(CHUNK))

    def loop_body(i, _):
        for b in (0, 1):
            c = 2 * i + b
            xcb, ecb, ocb = xc.at[b], ec.at[b], oc.at[b]

            @pl.when(i > 0)
            def _():
                oout(c - 2, b).wait()

            xin(c, b).wait()
            ein(c, b).wait()

            def p1(j, carry):
                ss, qq = carry
                for u in range(UNROLL):
                    sl = pl.ds((UNROLL * j + u) * L, L)
                    nss, nqq = [], []
                    for r in range(CHUNK):
                        h = xcb[r, sl] + ecb[r, sl]
                        ocb[r, sl] = h
                        nss.append(ss[r] + h)
                        nqq.append(qq[r] + h * h)
                    ss, qq = tuple(nss), tuple(nqq)
                return ss, qq

            ss, qq = lax.fori_loop(0, NJ // UNROLL, p1, (zeros, zeros))

            @pl.when(i < HALF - 1)
            def _():
                xin(c + 2, b).start()
                ein(c + 2, b).start()

            A, D = [], []
            for r in range(CHUNK):
                mv = _lane_sum(ss[r]) * (1.0 / F)
                var = _lane_sum(qq[r]) * (1.0 / F) - mv * mv + EPS
                rs = _rsqrt_newton(var)
                A.append(rs)
                D.append(mv * rs)

            def p2(j, _):
                for u in range(UNROLL):
                    sl = pl.ds((UNROLL * j + u) * L, L)
                    g = gv[sl]
                    bt = bv[sl]
                    for r in range(CHUNK):
                        h = ocb[r, sl]
                        ocb[r, sl] = (h * A[r] - D[r]) * g + bt
                return 0

            lax.fori_loop(0, NJ // UNROLL, p2, 0)
            oout(c, b).start()
        return 0

    lax.fori_loop(0, HALF, loop_body, 0)
    oout(NCH - 2, 0).wait()
    oout(NCH - 1, 1).wait()


_sc_kernel = functools.partial(
    pl.kernel,
    out_type=jax.ShapeDtypeStruct((R, F), jnp.float32),
    mesh=plsc.VectorSubcoreMesh(core_axis_name="c", subcore_axis_name="s"),
    scratch_types=[
        pltpu.VMEM((2, CHUNK, F), jnp.float32),
        pltpu.VMEM((2, CHUNK, F), jnp.float32),
        pltpu.VMEM((2, CHUNK, F), jnp.float32),
        pltpu.VMEM((F,), jnp.float32),
        pltpu.VMEM((F,), jnp.float32),
        pltpu.SemaphoreType.DMA,
        pltpu.SemaphoreType.DMA,
        pltpu.SemaphoreType.DMA,
        pltpu.SemaphoreType.DMA,
        pltpu.SemaphoreType.DMA,
        pltpu.SemaphoreType.DMA,
    ],
)(_sc_body)


def kernel(x, emb, gamma, beta):
    b, s, f = x.shape
    out = _sc_kernel(x.reshape(b * s, f), emb[:s], gamma, beta)
    return out.reshape(b, s, f)
